# baseline (speedup 1.0000x reference)
import functools

import jax
import jax.numpy as jnp
from jax import lax
from jax.experimental import pallas as pl
from jax.experimental.pallas import tpu as pltpu
from jax.experimental.pallas import tpu_sc as plsc

CH = 32


def _sc_add_kernel(B, S, H, NC, NS):
    NW = NC * NS
    rows_per_w = S // NW
    n_chunks = rows_per_w // CH
    mesh = plsc.VectorSubcoreMesh(core_axis_name="c", subcore_axis_name="s")

    @functools.partial(
        pl.kernel,
        mesh=mesh,
        out_type=jax.ShapeDtypeStruct((B, S, H), jnp.float32),
        scratch_types=[
            pltpu.VMEM((CH, H), jnp.float32),
            pltpu.VMEM((CH, H), jnp.float32),
        ],
    )
    def k(x_hbm, pe_hbm, out_hbm, pe_v, x_v):
        wid = lax.axis_index("s") * NC + lax.axis_index("c")
        seq0 = wid * rows_per_w

        def chunk_body(c, carry):
            base = seq0 + c * CH
            pltpu.sync_copy(pe_hbm.at[pl.ds(base, CH)], pe_v)
            for b in range(B):
                pltpu.sync_copy(x_hbm.at[b, pl.ds(base, CH)], x_v)

                def row_body(r, carry2):
                    for j in range(H // 16):
                        plsc.addupdate(
                            x_v.at[r, pl.ds(j * 16, 16)],
                            pe_v[r, pl.ds(j * 16, 16)],
                        )
                    return carry2

                lax.fori_loop(0, CH, row_body, 0)
                pltpu.sync_copy(x_v, out_hbm.at[b, pl.ds(base, CH)])
            return carry

        lax.fori_loop(0, n_chunks, chunk_body, 0)

    return k


def kernel(x, pe):
    B, S, H = x.shape
    info = plsc.get_sparse_core_info()
    k = _sc_add_kernel(B, S, H, info.num_cores, info.num_subcores)
    return k(x, pe)


# speedup vs baseline: 2.2165x; 2.2165x over previous
import functools

import jax
import jax.numpy as jnp
from jax import lax
from jax.experimental import pallas as pl
from jax.experimental.pallas import tpu as pltpu
from jax.experimental.pallas import tpu_sc as plsc

CH = 16


def _sc_add_kernel(B, S, H, NC, NS):
    NW = NC * NS
    rows_per_w = S // NW
    n_chunks = rows_per_w // CH
    mesh = plsc.VectorSubcoreMesh(core_axis_name="c", subcore_axis_name="s")

    @functools.partial(
        pl.kernel,
        mesh=mesh,
        out_type=jax.ShapeDtypeStruct((B, S, H), jnp.float32),
        scratch_types=[
            pltpu.VMEM((CH, H), jnp.float32),
            pltpu.VMEM((CH, H), jnp.float32),
            pltpu.VMEM((CH, H), jnp.float32),
            pltpu.VMEM((CH, H), jnp.float32),
            pltpu.VMEM((CH, H), jnp.float32),
            pltpu.SemaphoreType.DMA,
            pltpu.SemaphoreType.DMA,
            pltpu.SemaphoreType.DMA,
            pltpu.SemaphoreType.DMA,
        ],
    )
    def k(x_hbm, pe_hbm, out_hbm, pe_v, ib0, ib1, ob0, ob1, si0, si1, so0, so1):
        wid = lax.axis_index("s") * NC + lax.axis_index("c")
        seq0 = wid * rows_per_w
        ibs, obs, sis, sos = (ib0, ib1), (ob0, ob1), (si0, si1), (so0, so1)

        def x_src(b, base):
            return x_hbm.at[b, pl.ds(base, CH)]

        pltpu.async_copy(x_src(0, seq0), ib0, si0)
        pltpu.async_copy(x_src(1, seq0), ib1, si1)

        def chunk_body(c, carry):
            base = seq0 + c * CH
            pltpu.sync_copy(pe_hbm.at[pl.ds(base, CH)], pe_v)
            for b in range(B):
                p = b % 2
                ib, ob, si, so = ibs[p], obs[p], sis[p], sos[p]
                pltpu.make_async_copy(x_src(b, base), ib, si).wait()
                if b >= 2:
                    pltpu.make_async_copy(ob, out_hbm.at[b, pl.ds(base, CH)], so).wait()
                else:

                    @pl.when(c > 0)
                    def _():
                        pltpu.make_async_copy(
                            ob, out_hbm.at[b, pl.ds(base, CH)], so
                        ).wait()

                def row_body(r, carry2):
                    for j in range(H // 16):
                        sl = pl.ds(j * 16, 16)
                        ob[r, sl] = ib[r, sl] + pe_v[r, sl]
                    return carry2

                lax.fori_loop(0, CH, row_body, 0)

                if b + 2 < B:
                    nb, nbase = b + 2, base
                else:
                    nb = b + 2 - B
                    nbase = seq0 + jnp.minimum(c + 1, n_chunks - 1) * CH
                pltpu.async_copy(x_src(nb, nbase), ib, si)
                pltpu.async_copy(ob, out_hbm.at[b, pl.ds(base, CH)], so)
            return carry

        lax.fori_loop(0, n_chunks, chunk_body, 0)

        last = seq0 + (n_chunks - 1) * CH
        pltpu.make_async_copy(obs[0], out_hbm.at[B - 2, pl.ds(last, CH)], sos[0]).wait()
        pltpu.make_async_copy(obs[1], out_hbm.at[B - 1, pl.ds(last, CH)], sos[1]).wait()
        pltpu.make_async_copy(x_src(0, last), ibs[0], sis[0]).wait()
        pltpu.make_async_copy(x_src(1, last), ibs[1], sis[1]).wait()

    return k


def kernel(x, pe):
    B, S, H = x.shape
    info = plsc.get_sparse_core_info()
    k = _sc_add_kernel(B, S, H, info.num_cores, info.num_subcores)
    return k(x, pe)
